# baseline (speedup 1.0000x reference)
import functools

import jax
import jax.numpy as jnp
from jax import lax
from jax.experimental import pallas as pl
from jax.experimental.pallas import tpu as pltpu
from jax.experimental.pallas import tpu_sc as plsc

_NUM_CORES = 2
_NUM_SUBCORES = 16
_NUM_WORKERS = _NUM_CORES * _NUM_SUBCORES

_D = 128
_CHUNK = 160
_NBUF = 3


def _make_gather(n_total: int):
  n_per_w = n_total // _NUM_WORKERS
  assert n_per_w % _CHUNK == 0
  n_chunks = n_per_w // _CHUNK

  mesh = plsc.VectorSubcoreMesh(core_axis_name="c", subcore_axis_name="s")

  @functools.partial(
      pl.kernel,
      mesh=mesh,
      out_type=jax.ShapeDtypeStruct((n_total, _D), jnp.float32),
      scratch_types=(
          [pltpu.VMEM_SHARED((252, _D), jnp.float32)]
          + [pltpu.VMEM((_CHUNK,), jnp.int32) for _ in range(_NBUF)]
          + [pltpu.VMEM((_CHUNK, _D), jnp.float32) for _ in range(_NBUF)]
          + [pltpu.SemaphoreType.DMA for _ in range(3 * _NBUF)]
      ),
  )
  def gather_kernel(table_hbm, idx_hbm, out_hbm, table_sh, *scratch):
    idx_bufs = scratch[:_NBUF]
    row_bufs = scratch[_NBUF:2 * _NBUF]
    isems = scratch[2 * _NBUF:3 * _NBUF]
    gsems = scratch[3 * _NBUF:4 * _NBUF]
    osems = scratch[4 * _NBUF:5 * _NBUF]

    wid = lax.axis_index("s") * _NUM_CORES + lax.axis_index("c")
    base = wid * n_per_w

    def idx_copy(ch):
      b = ch % _NBUF
      return pltpu.make_async_copy(
          idx_hbm.at[pl.ds(base + ch * _CHUNK, _CHUNK)], idx_bufs[b],
          isems[b])

    def gather_copy(ch):
      b = ch % _NBUF
      return pltpu.make_async_copy(table_sh.at[idx_bufs[b]], row_bufs[b],
                                   gsems[b])

    def store_copy(ch):
      b = ch % _NBUF
      return pltpu.make_async_copy(
          row_bufs[b], out_hbm.at[pl.ds(base + ch * _CHUNK, _CHUNK)],
          osems[b])

    for b in range(_NBUF):
      idx_copy(b).start()

    @pl.when(lax.axis_index("s") == 0)
    def _():
      pltpu.sync_copy(table_hbm, table_sh)
    plsc.subcore_barrier()

    for ch in range(_NBUF - 1):
      idx_copy(ch).wait()
      gather_copy(ch).start()

    for ch in range(n_chunks):
      if ch >= 1:
        store_copy(ch - 1).wait()
      if ch + _NBUF - 1 < n_chunks:
        idx_copy(ch + _NBUF - 1).wait()
        gather_copy(ch + _NBUF - 1).start()
      gather_copy(ch).wait()
      store_copy(ch).start()
      if ch + _NBUF < n_chunks:
        idx_copy(ch + _NBUF).start()
    store_copy(n_chunks - 1).wait()

  return gather_kernel


def kernel(seq_idx, pos_embed):
  batch, seq = seq_idx.shape
  n_total = batch * seq
  idx_flat = seq_idx.reshape(n_total)
  out = _make_gather(n_total)(pos_embed, idx_flat)
  return out.reshape(batch, seq, _D)


# speedup vs baseline: 1.0230x; 1.0230x over previous
import functools

import jax
import jax.numpy as jnp
from jax import lax
from jax.experimental import pallas as pl
from jax.experimental.pallas import tpu as pltpu
from jax.experimental.pallas import tpu_sc as plsc

_NUM_CORES = 2
_NUM_SUBCORES = 16
_NUM_WORKERS = _NUM_CORES * _NUM_SUBCORES

_D = 128
_CHUNK = 320
_NBUF = 3


def _make_gather(n_total: int):
  n_per_w = n_total // _NUM_WORKERS
  assert n_per_w % _CHUNK == 0
  n_chunks = n_per_w // _CHUNK

  mesh = plsc.VectorSubcoreMesh(core_axis_name="c", subcore_axis_name="s")

  @functools.partial(
      pl.kernel,
      mesh=mesh,
      out_type=jax.ShapeDtypeStruct((n_total, _D), jnp.float32),
      scratch_types=(
          [pltpu.VMEM_SHARED((252, _D), jnp.float32)]
          + [pltpu.VMEM((_CHUNK,), jnp.int32) for _ in range(_NBUF)]
          + [pltpu.VMEM((_CHUNK, _D), jnp.float32) for _ in range(_NBUF)]
          + [pltpu.SemaphoreType.DMA for _ in range(3 * _NBUF)]
      ),
  )
  def gather_kernel(table_hbm, idx_hbm, out_hbm, table_sh, *scratch):
    idx_bufs = scratch[:_NBUF]
    row_bufs = scratch[_NBUF:2 * _NBUF]
    isems = scratch[2 * _NBUF:3 * _NBUF]
    gsems = scratch[3 * _NBUF:4 * _NBUF]
    osems = scratch[4 * _NBUF:5 * _NBUF]

    wid = lax.axis_index("s") * _NUM_CORES + lax.axis_index("c")
    base = wid * n_per_w

    def idx_copy(ch):
      b = ch % _NBUF
      return pltpu.make_async_copy(
          idx_hbm.at[pl.ds(base + ch * _CHUNK, _CHUNK)], idx_bufs[b],
          isems[b])

    def gather_copy(ch):
      b = ch % _NBUF
      return pltpu.make_async_copy(table_sh.at[idx_bufs[b]], row_bufs[b],
                                   gsems[b])

    def store_copy(ch):
      b = ch % _NBUF
      return pltpu.make_async_copy(
          row_bufs[b], out_hbm.at[pl.ds(base + ch * _CHUNK, _CHUNK)],
          osems[b])

    for b in range(_NBUF):
      idx_copy(b).start()

    @pl.when(lax.axis_index("s") == 0)
    def _():
      pltpu.sync_copy(table_hbm, table_sh)
    plsc.subcore_barrier()

    for ch in range(_NBUF - 1):
      idx_copy(ch).wait()
      gather_copy(ch).start()

    for ch in range(n_chunks):
      if ch >= 1:
        store_copy(ch - 1).wait()
      if ch + _NBUF - 1 < n_chunks:
        idx_copy(ch + _NBUF - 1).wait()
        gather_copy(ch + _NBUF - 1).start()
      gather_copy(ch).wait()
      store_copy(ch).start()
      if ch + _NBUF < n_chunks:
        idx_copy(ch + _NBUF).start()
    store_copy(n_chunks - 1).wait()

  return gather_kernel


def kernel(seq_idx, pos_embed):
  batch, seq = seq_idx.shape
  n_total = batch * seq
  idx_flat = seq_idx.reshape(n_total)
  out = _make_gather(n_total)(pos_embed, idx_flat)
  return out.reshape(batch, seq, _D)
